# baseline (speedup 1.0000x reference)
import functools
import math

import jax
import jax.numpy as jnp
from jax import lax
from jax.experimental import pallas as pl
from jax.experimental.pallas import tpu as pltpu
from jax.experimental.pallas import tpu_sc as plsc

N = 50000
E = 800000
D_IN = 128
H = 64
HH = 32
C_OUT = 40
L_LAYERS = 4
LAMDA = 0.5
ALPHA = 0.1

TRASH = N
ACC = 50048
ROWS_PER_TILE = 3128
RING = 5
GPC = 10
CHUNKS = 40
E_PAD = 819200
EROWS = E_PAD // 128

BR = 1000
GRID = N // BR


_sc_mesh = plsc.VectorSubcoreMesh(core_axis_name="c", subcore_axis_name="s")


@functools.partial(
    pl.kernel,
    out_type=jax.ShapeDtypeStruct((2, ACC, HH), jnp.float32),
    mesh=_sc_mesh,
    compiler_params=pltpu.CompilerParams(use_tc_tiling_on_sc=False),
    scratch_types=[
        pltpu.VMEM((2, GPC, 128), jnp.int32),
        pltpu.VMEM((2, GPC, 128), jnp.int32),
        pltpu.VMEM((2, GPC, 128), jnp.float32),
        pltpu.VMEM((RING, 128, HH), jnp.float32),
        pltpu.VMEM_SHARED((ACC, HH), jnp.float32),
        pltpu.SemaphoreType.DMA((2,)),
        pltpu.SemaphoreType.DMA((RING,)),
        pltpu.SemaphoreType.DMA((RING,)),
    ],
)
def _spmm_sc(tbl_hbm, src_hbm, dst_hbm, vals_hbm, zeros_hbm, out_hbm,
             src_v, dst_v, vals_v, rows_v, acc, isem, gsem, ssem):
    c = lax.axis_index("c")
    s = lax.axis_index("s")
    row_base = s * (CHUNKS * GPC)
    tbl = tbl_hbm.at[c]

    def _fire_idx(ci, buf):
        r0 = row_base + ci * GPC
        pltpu.async_copy(src_hbm.at[pl.ds(r0, GPC)], src_v.at[buf], isem.at[buf])
        pltpu.async_copy(dst_hbm.at[pl.ds(r0, GPC)], dst_v.at[buf], isem.at[buf])
        pltpu.async_copy(vals_hbm.at[pl.ds(r0, GPC)], vals_v.at[buf], isem.at[buf])

    def _wait_idx(buf):
        pltpu.make_async_copy(src_hbm.at[pl.ds(0, GPC)], src_v.at[buf], isem.at[buf]).wait()
        pltpu.make_async_copy(dst_hbm.at[pl.ds(0, GPC)], dst_v.at[buf], isem.at[buf]).wait()
        pltpu.make_async_copy(vals_hbm.at[pl.ds(0, GPC)], vals_v.at[buf], isem.at[buf]).wait()

    def _fire_gather(cbuf, g, slot):
        pltpu.async_copy(tbl.at[src_v.at[cbuf, g]], rows_v.at[slot],
                         gsem.at[slot])

    def _drain_gather(slot):
        pltpu.make_async_copy(tbl.at[pl.ds(0, 128)], rows_v.at[slot],
                              gsem.at[slot]).wait()

    def _scale(cbuf, g, slot):
        def _body(sub, carry):
            vv = vals_v[cbuf, g, pl.ds(sub * 16, 16)]
            for t in range(16):
                vb = jnp.full((16,), vv[t], jnp.float32)
                r = sub * 16 + t
                for j in range(HH // 16):
                    rows_v[slot, r, pl.ds(j * 16, 16)] = (
                        rows_v[slot, r, pl.ds(j * 16, 16)] * vb)
            return carry

        lax.fori_loop(0, 8, _body, 0)

    def _fire_scatter(cbuf, g, slot):
        pltpu.async_copy(rows_v.at[slot], acc.at[dst_v.at[cbuf, g]],
                         ssem.at[slot], add=True)

    def _drain_scatter(slot):
        pltpu.make_async_copy(rows_v.at[slot], acc.at[pl.ds(0, 128)],
                              ssem.at[slot]).wait()

    _fire_idx(0, 0)
    pltpu.sync_copy(zeros_hbm,
                    acc.at[pl.ds(s * ROWS_PER_TILE, ROWS_PER_TILE)])
    _wait_idx(0)
    plsc.subcore_barrier()

    TOTAL = CHUNKS * GPC

    def _step(t, carry):
        tin = t % GPC
        cchunk = t // GPC

        @pl.when((tin == RING) & (cchunk + 1 < CHUNKS) & (t < TOTAL))
        def _():
            _fire_idx(cchunk + 1, (cchunk + 1) % 2)

        @pl.when((tin == GPC - 1) & (cchunk + 1 < CHUNKS) & (t < TOTAL))
        def _():
            _wait_idx((cchunk + 1) % 2)

        @pl.when(t >= RING)
        def _():
            _drain_scatter(t % RING)

        @pl.when(t < TOTAL)
        def _():
            gch = (t + 0) // GPC
            _fire_gather((gch % 2), t % GPC, t % RING)

        @pl.when(t >= RING - 1)
        def _():
            gp = t - (RING - 1)
            pch = gp // GPC
            slot = gp % RING
            _drain_gather(slot)
            _scale(pch % 2, gp % GPC, slot)
            _fire_scatter(pch % 2, gp % GPC, slot)
        return carry

    lax.fori_loop(0, TOTAL + RING - 1, _step, 0)

    _drain_scatter((TOTAL - 1) % RING)
    plsc.subcore_barrier()

    pltpu.sync_copy(acc.at[pl.ds(s * ROWS_PER_TILE, ROWS_PER_TILE)],
                    out_hbm.at[c, pl.ds(s * ROWS_PER_TILE, ROWS_PER_TILE)])


def _prep_edges(ei_k, vals_k):
    dst = ei_k[0].astype(jnp.int32)
    src = ei_k[1].astype(jnp.int32)
    pad = E_PAD - E
    src_p = jnp.concatenate([src, jnp.zeros((pad,), jnp.int32)]).reshape(EROWS, 128)
    dst_p = jnp.concatenate([dst, jnp.full((pad,), TRASH, jnp.int32)]).reshape(EROWS, 128)
    vals_p = jnp.concatenate(
        [vals_k.astype(jnp.float32), jnp.zeros((pad,), jnp.float32)]
    ).reshape(EROWS, 128)
    return src_p, dst_p, vals_p


def _mm(a, b):
    return lax.dot_general(a, b, (((1,), (0,)), ((), ())),
                           preferred_element_type=jnp.float32)


def _split_store(split_ref, x):
    split_ref[0] = x[:, :HH]
    split_ref[1] = x[:, HH:]


_SPLIT_SPEC = pl.BlockSpec((2, BR, HH), lambda i: (0, i, 0))
_SPLIT_SHAPE = jax.ShapeDtypeStruct((2, N, HH), jnp.float32)
_FULL_SPEC = pl.BlockSpec((BR, H), lambda i: (i, 0))
_FULL_SHAPE = jax.ShapeDtypeStruct((N, H), jnp.float32)
_HI_SPEC = pl.BlockSpec((2, BR, HH), lambda i: (0, i, 0))


def _fc_body(x_ref, w_ref, b_ref, o_ref, os_ref):
    h = jax.nn.relu(_mm(x_ref[...], w_ref[...]) + b_ref[...])
    o_ref[...] = h
    _split_store(os_ref, h)


def _fc(x2, W, b):
    return pl.pallas_call(
        _fc_body,
        grid=(GRID,),
        in_specs=[
            pl.BlockSpec((BR, D_IN), lambda i: (i, 0)),
            pl.BlockSpec((D_IN, H), lambda i: (0, 0)),
            pl.BlockSpec((1, H), lambda i: (0, 0)),
        ],
        out_specs=[_FULL_SPEC, _SPLIT_SPEC],
        out_shape=[_FULL_SHAPE, _SPLIT_SHAPE],
    )(x2, W, b)


def _att_body(h_ref, t_ref, fw_ref, fb_ref, mf_ref, hatts_ref, w_ref):
    v = h_ref[...]
    T = t_ref[...]
    x1 = _mm(v, T)
    x2 = _mm(v, fw_ref[...]) + fb_ref[...]
    o1 = jax.nn.sigmoid(jnp.sum(x1, axis=1, keepdims=True) * x2)
    o2 = jax.nn.sigmoid(jnp.sum(x2, axis=1, keepdims=True) * x1)
    mf = mf_ref[0, 0]
    o = o1 * mf + o2 * (1.0 - mf)
    w = jax.nn.sigmoid(_mm(o, T))
    w_ref[...] = w
    _split_store(hatts_ref, v * w)


def _att(h, T, fw, fb, mf):
    return pl.pallas_call(
        _att_body,
        grid=(GRID,),
        in_specs=[
            pl.BlockSpec((BR, H), lambda i: (i, 0)),
            pl.BlockSpec((H, H), lambda i: (0, 0)),
            pl.BlockSpec((H, H), lambda i: (0, 0)),
            pl.BlockSpec((1, H), lambda i: (0, 0)),
            pl.BlockSpec((1, 1), lambda i: (0, 0)),
        ],
        out_specs=[_SPLIT_SPEC, _FULL_SPEC],
        out_shape=[_SPLIT_SHAPE, _FULL_SHAPE],
    )(h, T, fw, fb, mf)


def _support(hi_ref, h0_ref, m_ref):
    h0 = h0_ref[...]
    a = ALPHA * h0[:, :HH] + (1.0 - ALPHA) * hi_ref[0]
    b = ALPHA * h0[:, HH:] + (1.0 - ALPHA) * hi_ref[1]
    return _mm(a, m_ref[:HH, :]) + _mm(b, m_ref[HH:, :])


def _tail_body(hi_ref, h0_ref, m_ref, o_ref):
    o_ref[...] = jax.nn.relu(jnp.tanh(_support(hi_ref, h0_ref, m_ref)))


def _tail(hi2, h0, M):
    return pl.pallas_call(
        _tail_body,
        grid=(GRID,),
        in_specs=[
            pl.BlockSpec((2, BR, HH), lambda i: (0, i, 0)),
            pl.BlockSpec((BR, H), lambda i: (i, 0)),
            pl.BlockSpec((H, H), lambda i: (0, 0)),
        ],
        out_specs=_FULL_SPEC,
        out_shape=_FULL_SHAPE,
    )(hi2, h0, M)


def _tail_mul_body(hi_ref, h0_ref, m_ref, w_ref, o_ref, os_ref):
    h = jax.nn.relu(jnp.tanh(_support(hi_ref, h0_ref, m_ref))) * w_ref[...]
    o_ref[...] = h
    _split_store(os_ref, h)


def _tail_mul(hi2, h0, M, w):
    return pl.pallas_call(
        _tail_mul_body,
        grid=(GRID,),
        in_specs=[
            pl.BlockSpec((2, BR, HH), lambda i: (0, i, 0)),
            pl.BlockSpec((BR, H), lambda i: (i, 0)),
            pl.BlockSpec((H, H), lambda i: (0, 0)),
            pl.BlockSpec((BR, H), lambda i: (i, 0)),
        ],
        out_specs=[_FULL_SPEC, _SPLIT_SPEC],
        out_shape=[_FULL_SHAPE, _SPLIT_SHAPE],
    )(hi2, h0, M, w)


def _lsm(x):
    m = jnp.max(x, axis=1, keepdims=True)
    xm = x - m
    return xm - jnp.log(jnp.sum(jnp.exp(xm), axis=1, keepdims=True))


def _head_body(h0_ref, h1_ref, wo_ref, bo_ref, a_ref, b_ref, c_ref, d_ref):
    lo0 = _mm(h0_ref[...], wo_ref[...]) + bo_ref[...]
    lo1 = _mm(h1_ref[...], wo_ref[...]) + bo_ref[...]
    o0 = _lsm(lo0)
    o1 = _lsm(lo1)
    a_ref[...] = _lsm(lo0 + lo1)
    b_ref[...] = 0.5 * (o0 + o1)
    c_ref[...] = o0
    d_ref[...] = o1


def _head(h0f, h1f, Wo, bo):
    outs = jax.ShapeDtypeStruct((N, C_OUT), jnp.float32)
    return pl.pallas_call(
        _head_body,
        grid=(GRID,),
        in_specs=[
            pl.BlockSpec((BR, H), lambda i: (i, 0)),
            pl.BlockSpec((BR, H), lambda i: (i, 0)),
            pl.BlockSpec((H, C_OUT), lambda i: (0, 0)),
            pl.BlockSpec((1, C_OUT), lambda i: (0, 0)),
        ],
        out_specs=[pl.BlockSpec((BR, C_OUT), lambda i: (i, 0))] * 4,
        out_shape=[outs] * 4,
    )(h0f, h1f, Wo, bo)


def kernel(x, edge_index, adj_vals, nfeat_sum, W_fc, b_fc, W_out, b_out,
           W_conv, attn_conv_w, attn_fc_w, attn_fc_b, attn_mix_w):
    eye = jnp.eye(H, dtype=jnp.float32)
    Ms = []
    Ts = []
    for i in range(L_LAYERS):
        th = math.log(LAMDA / (i + 1) + 1.0)
        wtw = W_conv[i].T @ W_conv[i] + 1e-4 * eye
        Lc = jnp.linalg.cholesky(wtw)
        ortho = W_conv[i] @ jnp.linalg.inv(Lc).T
        Ms.append(th * ortho + (1.0 - th) * eye)
        cw = attn_conv_w[i]
        Ts.append(cw[1] * eye + cw[0] * jnp.eye(H, k=1, dtype=jnp.float32)
                  + cw[2] * jnp.eye(H, k=-1, dtype=jnp.float32))
    mfs = jax.nn.sigmoid(attn_mix_w).reshape(L_LAYERS, 1, 1)

    src0, dst0, vals0 = _prep_edges(edge_index[0], adj_vals[0])
    src1, dst1, vals1 = _prep_edges(edge_index[1], adj_vals[1])
    zeros = jnp.zeros((ROWS_PER_TILE, HH), jnp.float32)
    h0, _ = _fc(x[0], W_fc[0], b_fc[0].reshape(1, H))
    h1, h1_split = _fc(x[1], W_fc[1], b_fc[1].reshape(1, H))
    anchor0, anchor1 = h0, h1
    for i in range(L_LAYERS):
        h0att_split, w_i = _att(h0, Ts[i], attn_fc_w[i],
                                attn_fc_b[i].reshape(1, H), mfs[i])
        hi0 = _spmm_sc(h0att_split, src0, dst0, vals0, zeros)
        hi1 = _spmm_sc(h1_split, src1, dst1, vals1, zeros)
        h0 = _tail(hi0, anchor0, Ms[i])
        h1, h1_split = _tail_mul(hi1, anchor1, Ms[i], w_i)

    return _head(h0, h1, W_out, b_out.reshape(1, C_OUT))


# speedup vs baseline: 1.0395x; 1.0395x over previous
import functools
import math

import jax
import jax.numpy as jnp
from jax import lax
from jax.experimental import pallas as pl
from jax.experimental.pallas import tpu as pltpu
from jax.experimental.pallas import tpu_sc as plsc

N = 50000
E = 800000
D_IN = 128
H = 64
HH = 32
C_OUT = 40
L_LAYERS = 4
LAMDA = 0.5
ALPHA = 0.1

TRASH = N
ACC = 50048
ROWS_PER_TILE = 3128
ZROWS = 184
RING = 4
GPC = 10
CHUNKS = 40
E_PAD = 819200
EROWS = E_PAD // 128

BR = 1000
GRID = N // BR


_sc_mesh = plsc.VectorSubcoreMesh(core_axis_name="c", subcore_axis_name="s")


@functools.partial(
    pl.kernel,
    out_type=jax.ShapeDtypeStruct((2, ACC, HH), jnp.float32),
    mesh=_sc_mesh,
    compiler_params=pltpu.CompilerParams(use_tc_tiling_on_sc=False),
    scratch_types=[
        pltpu.VMEM((2, GPC, 128), jnp.int32),
        pltpu.VMEM((2, GPC, 128), jnp.int32),
        pltpu.VMEM((2, GPC, 128), jnp.float32),
        pltpu.VMEM((RING, 128, HH), jnp.float32),
        pltpu.VMEM((ZROWS, HH), jnp.float32),
        pltpu.VMEM_SHARED((ACC, HH), jnp.float32),
        pltpu.SemaphoreType.DMA((2,)),
        pltpu.SemaphoreType.DMA((RING,)),
        pltpu.SemaphoreType.DMA((RING,)),
    ],
)
def _spmm_sc(tbl_hbm, src_hbm, dst_hbm, vals_hbm, out_hbm,
             src_v, dst_v, vals_v, rows_v, zb_v, acc, isem, gsem, ssem):
    c = lax.axis_index("c")
    s = lax.axis_index("s")
    row_base = s * (CHUNKS * GPC)
    tbl = tbl_hbm.at[c]

    def _fire_idx(ci, buf):
        r0 = row_base + ci * GPC
        pltpu.async_copy(src_hbm.at[pl.ds(r0, GPC)], src_v.at[buf], isem.at[buf])
        pltpu.async_copy(dst_hbm.at[pl.ds(r0, GPC)], dst_v.at[buf], isem.at[buf])
        pltpu.async_copy(vals_hbm.at[pl.ds(r0, GPC)], vals_v.at[buf], isem.at[buf])

    def _wait_idx(buf):
        pltpu.make_async_copy(src_hbm.at[pl.ds(0, GPC)], src_v.at[buf], isem.at[buf]).wait()
        pltpu.make_async_copy(dst_hbm.at[pl.ds(0, GPC)], dst_v.at[buf], isem.at[buf]).wait()
        pltpu.make_async_copy(vals_hbm.at[pl.ds(0, GPC)], vals_v.at[buf], isem.at[buf]).wait()

    def _fire_gather(cbuf, g, slot):
        pltpu.async_copy(tbl.at[src_v.at[cbuf, g]], rows_v.at[slot],
                         gsem.at[slot])

    def _drain_gather(slot):
        pltpu.make_async_copy(tbl.at[pl.ds(0, 128)], rows_v.at[slot],
                              gsem.at[slot]).wait()

    def _scale(cbuf, g, slot):
        def _body(sub, carry):
            vv = vals_v[cbuf, g, pl.ds(sub * 16, 16)]
            for t in range(16):
                vb = jnp.full((16,), vv[t], jnp.float32)
                r = sub * 16 + t
                for j in range(HH // 16):
                    rows_v[slot, r, pl.ds(j * 16, 16)] = (
                        rows_v[slot, r, pl.ds(j * 16, 16)] * vb)
            return carry

        lax.fori_loop(0, 8, _body, 0)

    def _fire_scatter(cbuf, g, slot):
        pltpu.async_copy(rows_v.at[slot], acc.at[dst_v.at[cbuf, g]],
                         ssem.at[slot], add=True)

    def _drain_scatter(slot):
        pltpu.make_async_copy(rows_v.at[slot], acc.at[pl.ds(0, 128)],
                              ssem.at[slot]).wait()

    _fire_idx(0, 0)

    z16 = jnp.zeros((16,), jnp.float32)

    def _zrow(r, carry):
        for j in range(HH // 16):
            zb_v[r, pl.ds(j * 16, 16)] = z16
        return carry

    lax.fori_loop(0, ZROWS, _zrow, 0)

    def _zacc(t, carry):
        pltpu.sync_copy(zb_v, acc.at[pl.ds(s * ROWS_PER_TILE + t * ZROWS, ZROWS)])
        return carry

    lax.fori_loop(0, ROWS_PER_TILE // ZROWS, _zacc, 0)

    _wait_idx(0)
    plsc.subcore_barrier()

    TOTAL = CHUNKS * GPC

    def _step(t, carry):
        tin = t % GPC
        cchunk = t // GPC

        @pl.when((tin == RING) & (cchunk + 1 < CHUNKS) & (t < TOTAL))
        def _():
            _fire_idx(cchunk + 1, (cchunk + 1) % 2)

        @pl.when((tin == GPC - 1) & (cchunk + 1 < CHUNKS) & (t < TOTAL))
        def _():
            _wait_idx((cchunk + 1) % 2)

        @pl.when(t >= RING - 1)
        def _():
            gp = t - (RING - 1)
            pch = gp // GPC
            slot = gp % RING
            _drain_gather(slot)
            _scale(pch % 2, gp % GPC, slot)
            _fire_scatter(pch % 2, gp % GPC, slot)

        @pl.when(t >= RING)
        def _():
            _drain_scatter(t % RING)

        @pl.when(t < TOTAL)
        def _():
            gch = t // GPC
            _fire_gather((gch % 2), t % GPC, t % RING)
        return carry

    lax.fori_loop(0, TOTAL + RING - 1, _step, 0)

    _drain_scatter((TOTAL - 1) % RING)
    plsc.subcore_barrier()

    pltpu.sync_copy(acc.at[pl.ds(s * ROWS_PER_TILE, ROWS_PER_TILE)],
                    out_hbm.at[c, pl.ds(s * ROWS_PER_TILE, ROWS_PER_TILE)])


def _prep_edges(ei_k, vals_k):
    dst = ei_k[0].astype(jnp.int32)
    src = ei_k[1].astype(jnp.int32)
    pad = E_PAD - E
    src_p = jnp.concatenate([src, jnp.zeros((pad,), jnp.int32)]).reshape(EROWS, 128)
    dst_p = jnp.concatenate([dst, jnp.full((pad,), TRASH, jnp.int32)]).reshape(EROWS, 128)
    vals_p = jnp.concatenate(
        [vals_k.astype(jnp.float32), jnp.zeros((pad,), jnp.float32)]
    ).reshape(EROWS, 128)
    return src_p, dst_p, vals_p


def _mm(a, b):
    return lax.dot_general(a, b, (((1,), (0,)), ((), ())),
                           preferred_element_type=jnp.float32)


def _split_store(split_ref, x):
    split_ref[0] = x[:, :HH]
    split_ref[1] = x[:, HH:]


_SPLIT_SPEC = pl.BlockSpec((2, BR, HH), lambda i: (0, i, 0))
_SPLIT_SHAPE = jax.ShapeDtypeStruct((2, N, HH), jnp.float32)
_FULL_SPEC = pl.BlockSpec((BR, H), lambda i: (i, 0))
_FULL_SHAPE = jax.ShapeDtypeStruct((N, H), jnp.float32)
_HI_SPEC = pl.BlockSpec((2, BR, HH), lambda i: (0, i, 0))


def _fc_body(x_ref, w_ref, b_ref, o_ref, os_ref):
    h = jax.nn.relu(_mm(x_ref[...], w_ref[...]) + b_ref[...])
    o_ref[...] = h
    _split_store(os_ref, h)


def _fc(x2, W, b):
    return pl.pallas_call(
        _fc_body,
        grid=(GRID,),
        in_specs=[
            pl.BlockSpec((BR, D_IN), lambda i: (i, 0)),
            pl.BlockSpec((D_IN, H), lambda i: (0, 0)),
            pl.BlockSpec((1, H), lambda i: (0, 0)),
        ],
        out_specs=[_FULL_SPEC, _SPLIT_SPEC],
        out_shape=[_FULL_SHAPE, _SPLIT_SHAPE],
    )(x2, W, b)


def _att_body(h_ref, t_ref, fw_ref, fb_ref, mf_ref, hatts_ref, w_ref):
    v = h_ref[...]
    T = t_ref[...]
    x1 = _mm(v, T)
    x2 = _mm(v, fw_ref[...]) + fb_ref[...]
    o1 = jax.nn.sigmoid(jnp.sum(x1, axis=1, keepdims=True) * x2)
    o2 = jax.nn.sigmoid(jnp.sum(x2, axis=1, keepdims=True) * x1)
    mf = mf_ref[0, 0]
    o = o1 * mf + o2 * (1.0 - mf)
    w = jax.nn.sigmoid(_mm(o, T))
    w_ref[...] = w
    _split_store(hatts_ref, v * w)


def _att(h, T, fw, fb, mf):
    return pl.pallas_call(
        _att_body,
        grid=(GRID,),
        in_specs=[
            pl.BlockSpec((BR, H), lambda i: (i, 0)),
            pl.BlockSpec((H, H), lambda i: (0, 0)),
            pl.BlockSpec((H, H), lambda i: (0, 0)),
            pl.BlockSpec((1, H), lambda i: (0, 0)),
            pl.BlockSpec((1, 1), lambda i: (0, 0)),
        ],
        out_specs=[_SPLIT_SPEC, _FULL_SPEC],
        out_shape=[_SPLIT_SHAPE, _FULL_SHAPE],
    )(h, T, fw, fb, mf)


def _support(hi_ref, h0_ref, m_ref):
    h0 = h0_ref[...]
    a = ALPHA * h0[:, :HH] + (1.0 - ALPHA) * hi_ref[0]
    b = ALPHA * h0[:, HH:] + (1.0 - ALPHA) * hi_ref[1]
    return _mm(a, m_ref[:HH, :]) + _mm(b, m_ref[HH:, :])


def _tail_body(hi_ref, h0_ref, m_ref, o_ref):
    o_ref[...] = jax.nn.relu(jnp.tanh(_support(hi_ref, h0_ref, m_ref)))


def _tail(hi2, h0, M):
    return pl.pallas_call(
        _tail_body,
        grid=(GRID,),
        in_specs=[
            pl.BlockSpec((2, BR, HH), lambda i: (0, i, 0)),
            pl.BlockSpec((BR, H), lambda i: (i, 0)),
            pl.BlockSpec((H, H), lambda i: (0, 0)),
        ],
        out_specs=_FULL_SPEC,
        out_shape=_FULL_SHAPE,
    )(hi2, h0, M)


def _tail_mul_body(hi_ref, h0_ref, m_ref, w_ref, o_ref, os_ref):
    h = jax.nn.relu(jnp.tanh(_support(hi_ref, h0_ref, m_ref))) * w_ref[...]
    o_ref[...] = h
    _split_store(os_ref, h)


def _tail_mul(hi2, h0, M, w):
    return pl.pallas_call(
        _tail_mul_body,
        grid=(GRID,),
        in_specs=[
            pl.BlockSpec((2, BR, HH), lambda i: (0, i, 0)),
            pl.BlockSpec((BR, H), lambda i: (i, 0)),
            pl.BlockSpec((H, H), lambda i: (0, 0)),
            pl.BlockSpec((BR, H), lambda i: (i, 0)),
        ],
        out_specs=[_FULL_SPEC, _SPLIT_SPEC],
        out_shape=[_FULL_SHAPE, _SPLIT_SHAPE],
    )(hi2, h0, M, w)


def _lsm(x):
    m = jnp.max(x, axis=1, keepdims=True)
    xm = x - m
    return xm - jnp.log(jnp.sum(jnp.exp(xm), axis=1, keepdims=True))


def _head_body(h0_ref, h1_ref, wo_ref, bo_ref, a_ref, b_ref, c_ref, d_ref):
    lo0 = _mm(h0_ref[...], wo_ref[...]) + bo_ref[...]
    lo1 = _mm(h1_ref[...], wo_ref[...]) + bo_ref[...]
    o0 = _lsm(lo0)
    o1 = _lsm(lo1)
    a_ref[...] = _lsm(lo0 + lo1)
    b_ref[...] = 0.5 * (o0 + o1)
    c_ref[...] = o0
    d_ref[...] = o1


def _head(h0f, h1f, Wo, bo):
    outs = jax.ShapeDtypeStruct((N, C_OUT), jnp.float32)
    return pl.pallas_call(
        _head_body,
        grid=(GRID,),
        in_specs=[
            pl.BlockSpec((BR, H), lambda i: (i, 0)),
            pl.BlockSpec((BR, H), lambda i: (i, 0)),
            pl.BlockSpec((H, C_OUT), lambda i: (0, 0)),
            pl.BlockSpec((1, C_OUT), lambda i: (0, 0)),
        ],
        out_specs=[pl.BlockSpec((BR, C_OUT), lambda i: (i, 0))] * 4,
        out_shape=[outs] * 4,
    )(h0f, h1f, Wo, bo)


def kernel(x, edge_index, adj_vals, nfeat_sum, W_fc, b_fc, W_out, b_out,
           W_conv, attn_conv_w, attn_fc_w, attn_fc_b, attn_mix_w):
    eye = jnp.eye(H, dtype=jnp.float32)
    Ms = []
    Ts = []
    for i in range(L_LAYERS):
        th = math.log(LAMDA / (i + 1) + 1.0)
        wtw = W_conv[i].T @ W_conv[i] + 1e-4 * eye
        Lc = jnp.linalg.cholesky(wtw)
        ortho = W_conv[i] @ jnp.linalg.inv(Lc).T
        Ms.append(th * ortho + (1.0 - th) * eye)
        cw = attn_conv_w[i]
        Ts.append(cw[1] * eye + cw[0] * jnp.eye(H, k=1, dtype=jnp.float32)
                  + cw[2] * jnp.eye(H, k=-1, dtype=jnp.float32))
    mfs = jax.nn.sigmoid(attn_mix_w).reshape(L_LAYERS, 1, 1)

    src0, dst0, vals0 = _prep_edges(edge_index[0], adj_vals[0])
    src1, dst1, vals1 = _prep_edges(edge_index[1], adj_vals[1])
    h0, _ = _fc(x[0], W_fc[0], b_fc[0].reshape(1, H))
    h1, h1_split = _fc(x[1], W_fc[1], b_fc[1].reshape(1, H))
    anchor0, anchor1 = h0, h1
    for i in range(L_LAYERS):
        h0att_split, w_i = _att(h0, Ts[i], attn_fc_w[i],
                                attn_fc_b[i].reshape(1, H), mfs[i])
        hi0 = _spmm_sc(h0att_split, src0, dst0, vals0)
        hi1 = _spmm_sc(h1_split, src1, dst1, vals1)
        h0 = _tail(hi0, anchor0, Ms[i])
        h1, h1_split = _tail_mul(hi1, anchor1, Ms[i], w_i)

    return _head(h0, h1, W_out, b_out.reshape(1, C_OUT))


# speedup vs baseline: 1.1004x; 1.0586x over previous
import functools
import math

import jax
import jax.numpy as jnp
from jax import lax
from jax.experimental import pallas as pl
from jax.experimental.pallas import tpu as pltpu
from jax.experimental.pallas import tpu_sc as plsc

N = 50000
E = 800000
D_IN = 128
H = 64
HH = 32
C_OUT = 40
L_LAYERS = 4
LAMDA = 0.5
ALPHA = 0.1

TRASH = N
ACC = 50048
ROWS_PER_TILE = 3128
ZROWS = 184
RING = 4
GPC = 10
CHUNKS = 40
E_PAD = 819200
EROWS = E_PAD // 128

BR = 1000
GRID = N // BR


_sc_mesh = plsc.VectorSubcoreMesh(core_axis_name="c", subcore_axis_name="s")


@functools.partial(
    pl.kernel,
    out_type=jax.ShapeDtypeStruct((2, ACC, HH), jnp.float32),
    mesh=_sc_mesh,
    compiler_params=pltpu.CompilerParams(use_tc_tiling_on_sc=False),
    scratch_types=[
        pltpu.VMEM((2, GPC, 128), jnp.int32),
        pltpu.VMEM((2, GPC, 128), jnp.int32),
        pltpu.VMEM((2, GPC, 128), jnp.float32),
        pltpu.VMEM((RING, 128, HH), jnp.float32),
        pltpu.VMEM((ZROWS, HH), jnp.float32),
        pltpu.VMEM_SHARED((ACC, HH), jnp.float32),
        pltpu.SemaphoreType.DMA((2,)),
        pltpu.SemaphoreType.DMA((RING,)),
        pltpu.SemaphoreType.DMA((RING,)),
    ],
)
def _spmm_sc(tbl_hbm, src_hbm, dst_hbm, vals_hbm, out_hbm,
             src_v, dst_v, vals_v, rows_v, zb_v, acc, isem, gsem, ssem):
    c = lax.axis_index("c")
    s = lax.axis_index("s")
    row_base = s * (CHUNKS * GPC)
    tbl = tbl_hbm.at[c]

    def _fire_idx(ci, buf):
        r0 = row_base + ci * GPC
        pltpu.async_copy(src_hbm.at[pl.ds(r0, GPC)], src_v.at[buf], isem.at[buf])
        pltpu.async_copy(dst_hbm.at[pl.ds(r0, GPC)], dst_v.at[buf], isem.at[buf])
        pltpu.async_copy(vals_hbm.at[pl.ds(r0, GPC)], vals_v.at[buf], isem.at[buf])

    def _wait_idx(buf):
        pltpu.make_async_copy(src_hbm.at[pl.ds(0, GPC)], src_v.at[buf], isem.at[buf]).wait()
        pltpu.make_async_copy(dst_hbm.at[pl.ds(0, GPC)], dst_v.at[buf], isem.at[buf]).wait()
        pltpu.make_async_copy(vals_hbm.at[pl.ds(0, GPC)], vals_v.at[buf], isem.at[buf]).wait()

    def _fire_gather(cbuf, g, slot):
        pltpu.async_copy(tbl.at[src_v.at[cbuf, g]], rows_v.at[slot],
                         gsem.at[slot])

    def _drain_gather(slot):
        pltpu.make_async_copy(tbl.at[pl.ds(0, 128)], rows_v.at[slot],
                              gsem.at[slot]).wait()

    def _scale(cbuf, g, slot):
        def _body(sub, carry):
            vv = vals_v[cbuf, g, pl.ds(sub * 16, 16)]
            for t in range(16):
                vb = jnp.full((16,), vv[t], jnp.float32)
                r = sub * 16 + t
                for j in range(HH // 16):
                    rows_v[slot, r, pl.ds(j * 16, 16)] = (
                        rows_v[slot, r, pl.ds(j * 16, 16)] * vb)
            return carry

        lax.fori_loop(0, 8, _body, 0)

    def _fire_scatter(cbuf, g, slot):
        pltpu.async_copy(rows_v.at[slot], acc.at[dst_v.at[cbuf, g]],
                         ssem.at[slot], add=True)

    def _drain_scatter(slot):
        pltpu.make_async_copy(rows_v.at[slot], acc.at[pl.ds(0, 128)],
                              ssem.at[slot]).wait()

    _fire_idx(0, 0)

    z16 = jnp.zeros((16,), jnp.float32)

    def _zrow(r, carry):
        for j in range(HH // 16):
            zb_v[r, pl.ds(j * 16, 16)] = z16
        return carry

    lax.fori_loop(0, ZROWS, _zrow, 0)

    def _zacc(t, carry):
        pltpu.sync_copy(zb_v, acc.at[pl.ds(s * ROWS_PER_TILE + t * ZROWS, ZROWS)])
        return carry

    lax.fori_loop(0, ROWS_PER_TILE // ZROWS, _zacc, 0)

    _wait_idx(0)
    plsc.subcore_barrier()

    TOTAL = CHUNKS * GPC

    def _step(t, carry):
        tin = t % GPC
        cchunk = t // GPC

        @pl.when((tin == RING) & (cchunk + 1 < CHUNKS) & (t < TOTAL))
        def _():
            _fire_idx(cchunk + 1, (cchunk + 1) % 2)

        @pl.when((tin == GPC - 1) & (cchunk + 1 < CHUNKS) & (t < TOTAL))
        def _():
            _wait_idx((cchunk + 1) % 2)

        @pl.when(t >= RING - 1)
        def _():
            gp = t - (RING - 1)
            pch = gp // GPC
            slot = gp % RING
            _drain_gather(slot)
            _scale(pch % 2, gp % GPC, slot)
            _fire_scatter(pch % 2, gp % GPC, slot)

        @pl.when(t >= RING)
        def _():
            _drain_scatter(t % RING)

        @pl.when(t < TOTAL)
        def _():
            gch = t // GPC
            _fire_gather((gch % 2), t % GPC, t % RING)
        return carry

    lax.fori_loop(0, TOTAL + RING - 1, _step, 0)

    _drain_scatter((TOTAL - 1) % RING)
    plsc.subcore_barrier()

    pltpu.sync_copy(acc.at[pl.ds(s * ROWS_PER_TILE, ROWS_PER_TILE)],
                    out_hbm.at[c, pl.ds(s * ROWS_PER_TILE, ROWS_PER_TILE)])


def _prep_edges(ei_k, vals_k):
    dst = ei_k[0].astype(jnp.int32)
    src = ei_k[1].astype(jnp.int32)
    pad = E_PAD - E
    src_p = jnp.concatenate([src, jnp.zeros((pad,), jnp.int32)]).reshape(EROWS, 128)
    dst_p = jnp.concatenate([dst, jnp.full((pad,), TRASH, jnp.int32)]).reshape(EROWS, 128)
    vals_p = jnp.concatenate(
        [vals_k.astype(jnp.float32), jnp.zeros((pad,), jnp.float32)]
    ).reshape(EROWS, 128)
    return src_p, dst_p, vals_p


def _mm(a, b):
    return lax.dot_general(a, b, (((1,), (0,)), ((), ())),
                           preferred_element_type=jnp.float32)


def _split_store(split_ref, x):
    split_ref[0] = x[:, :HH]
    split_ref[1] = x[:, HH:]


_SPLIT_SPEC = pl.BlockSpec((2, BR, HH), lambda i: (0, i, 0))
_SPLIT_SHAPE = jax.ShapeDtypeStruct((2, N, HH), jnp.float32)
_FULL_SPEC = pl.BlockSpec((BR, H), lambda i: (i, 0))
_FULL_SHAPE = jax.ShapeDtypeStruct((N, H), jnp.float32)
_HI_SPEC = pl.BlockSpec((2, BR, HH), lambda i: (0, i, 0))


def _fc_body(x_ref, w_ref, b_ref, o_ref, os_ref):
    h = jax.nn.relu(_mm(x_ref[...], w_ref[...]) + b_ref[...])
    o_ref[...] = h
    _split_store(os_ref, h)


def _fc(x2, W, b):
    return pl.pallas_call(
        _fc_body,
        grid=(GRID,),
        in_specs=[
            pl.BlockSpec((BR, D_IN), lambda i: (i, 0)),
            pl.BlockSpec((D_IN, H), lambda i: (0, 0)),
            pl.BlockSpec((1, H), lambda i: (0, 0)),
        ],
        out_specs=[_FULL_SPEC, _SPLIT_SPEC],
        out_shape=[_FULL_SHAPE, _SPLIT_SHAPE],
    )(x2, W, b)


def _att_core(v, T, fw, fb, mf):
    x1 = _mm(v, T)
    x2 = _mm(v, fw) + fb
    o1 = jax.nn.sigmoid(jnp.sum(x1, axis=1, keepdims=True) * x2)
    o2 = jax.nn.sigmoid(jnp.sum(x2, axis=1, keepdims=True) * x1)
    o = o1 * mf + o2 * (1.0 - mf)
    return jax.nn.sigmoid(_mm(o, T))


_ATT_W_SPECS = [
    pl.BlockSpec((H, H), lambda i: (0, 0)),
    pl.BlockSpec((H, H), lambda i: (0, 0)),
    pl.BlockSpec((1, H), lambda i: (0, 0)),
    pl.BlockSpec((1, 1), lambda i: (0, 0)),
]


def _fc_att_body(x_ref, w_ref, b_ref, t_ref, fw_ref, fb_ref, mf_ref,
                 h_ref, hatts_ref, wout_ref):
    h = jax.nn.relu(_mm(x_ref[...], w_ref[...]) + b_ref[...])
    h_ref[...] = h
    w = _att_core(h, t_ref[...], fw_ref[...], fb_ref[...], mf_ref[0, 0])
    wout_ref[...] = w
    _split_store(hatts_ref, h * w)


def _fc_att(x2, W, b, T, fw, fb, mf):
    return pl.pallas_call(
        _fc_att_body,
        grid=(GRID,),
        in_specs=[
            pl.BlockSpec((BR, D_IN), lambda i: (i, 0)),
            pl.BlockSpec((D_IN, H), lambda i: (0, 0)),
            pl.BlockSpec((1, H), lambda i: (0, 0)),
        ] + _ATT_W_SPECS,
        out_specs=[_FULL_SPEC, _SPLIT_SPEC, _FULL_SPEC],
        out_shape=[_FULL_SHAPE, _SPLIT_SHAPE, _FULL_SHAPE],
    )(x2, W, b, T, fw, fb, mf)


def _tail_att_body(hi_ref, h0_ref, m_ref, t_ref, fw_ref, fb_ref, mf_ref,
                   hatts_ref, wout_ref):
    h = jax.nn.relu(jnp.tanh(_support(hi_ref, h0_ref, m_ref)))
    w = _att_core(h, t_ref[...], fw_ref[...], fb_ref[...], mf_ref[0, 0])
    wout_ref[...] = w
    _split_store(hatts_ref, h * w)


def _tail_att(hi2, h0, M, T, fw, fb, mf):
    return pl.pallas_call(
        _tail_att_body,
        grid=(GRID,),
        in_specs=[
            pl.BlockSpec((2, BR, HH), lambda i: (0, i, 0)),
            pl.BlockSpec((BR, H), lambda i: (i, 0)),
            pl.BlockSpec((H, H), lambda i: (0, 0)),
        ] + _ATT_W_SPECS,
        out_specs=[_SPLIT_SPEC, _FULL_SPEC],
        out_shape=[_SPLIT_SHAPE, _FULL_SHAPE],
    )(hi2, h0, M, T, fw, fb, mf)


def _support(hi_ref, h0_ref, m_ref):
    h0 = h0_ref[...]
    a = ALPHA * h0[:, :HH] + (1.0 - ALPHA) * hi_ref[0]
    b = ALPHA * h0[:, HH:] + (1.0 - ALPHA) * hi_ref[1]
    return _mm(a, m_ref[:HH, :]) + _mm(b, m_ref[HH:, :])


def _tail_body(hi_ref, h0_ref, m_ref, o_ref):
    o_ref[...] = jax.nn.relu(jnp.tanh(_support(hi_ref, h0_ref, m_ref)))


def _tail(hi2, h0, M):
    return pl.pallas_call(
        _tail_body,
        grid=(GRID,),
        in_specs=[
            pl.BlockSpec((2, BR, HH), lambda i: (0, i, 0)),
            pl.BlockSpec((BR, H), lambda i: (i, 0)),
            pl.BlockSpec((H, H), lambda i: (0, 0)),
        ],
        out_specs=_FULL_SPEC,
        out_shape=_FULL_SHAPE,
    )(hi2, h0, M)


def _tail_mul_body(hi_ref, h0_ref, m_ref, w_ref, o_ref, os_ref):
    h = jax.nn.relu(jnp.tanh(_support(hi_ref, h0_ref, m_ref))) * w_ref[...]
    o_ref[...] = h
    _split_store(os_ref, h)


def _tail_mul(hi2, h0, M, w):
    return pl.pallas_call(
        _tail_mul_body,
        grid=(GRID,),
        in_specs=[
            pl.BlockSpec((2, BR, HH), lambda i: (0, i, 0)),
            pl.BlockSpec((BR, H), lambda i: (i, 0)),
            pl.BlockSpec((H, H), lambda i: (0, 0)),
            pl.BlockSpec((BR, H), lambda i: (i, 0)),
        ],
        out_specs=[_FULL_SPEC, _SPLIT_SPEC],
        out_shape=[_FULL_SHAPE, _SPLIT_SHAPE],
    )(hi2, h0, M, w)


def _lsm(x):
    m = jnp.max(x, axis=1, keepdims=True)
    xm = x - m
    return xm - jnp.log(jnp.sum(jnp.exp(xm), axis=1, keepdims=True))


def _head_body(h0_ref, h1_ref, wo_ref, bo_ref, a_ref, b_ref, c_ref, d_ref):
    lo0 = _mm(h0_ref[...], wo_ref[...]) + bo_ref[...]
    lo1 = _mm(h1_ref[...], wo_ref[...]) + bo_ref[...]
    o0 = _lsm(lo0)
    o1 = _lsm(lo1)
    a_ref[...] = _lsm(lo0 + lo1)
    b_ref[...] = 0.5 * (o0 + o1)
    c_ref[...] = o0
    d_ref[...] = o1


def _head(h0f, h1f, Wo, bo):
    outs = jax.ShapeDtypeStruct((N, C_OUT), jnp.float32)
    return pl.pallas_call(
        _head_body,
        grid=(GRID,),
        in_specs=[
            pl.BlockSpec((BR, H), lambda i: (i, 0)),
            pl.BlockSpec((BR, H), lambda i: (i, 0)),
            pl.BlockSpec((H, C_OUT), lambda i: (0, 0)),
            pl.BlockSpec((1, C_OUT), lambda i: (0, 0)),
        ],
        out_specs=[pl.BlockSpec((BR, C_OUT), lambda i: (i, 0))] * 4,
        out_shape=[outs] * 4,
    )(h0f, h1f, Wo, bo)


def kernel(x, edge_index, adj_vals, nfeat_sum, W_fc, b_fc, W_out, b_out,
           W_conv, attn_conv_w, attn_fc_w, attn_fc_b, attn_mix_w):
    eye = jnp.eye(H, dtype=jnp.float32)
    Ms = []
    Ts = []
    for i in range(L_LAYERS):
        th = math.log(LAMDA / (i + 1) + 1.0)
        wtw = W_conv[i].T @ W_conv[i] + 1e-4 * eye
        Lc = jnp.linalg.cholesky(wtw)
        ortho = W_conv[i] @ jnp.linalg.inv(Lc).T
        Ms.append(th * ortho + (1.0 - th) * eye)
        cw = attn_conv_w[i]
        Ts.append(cw[1] * eye + cw[0] * jnp.eye(H, k=1, dtype=jnp.float32)
                  + cw[2] * jnp.eye(H, k=-1, dtype=jnp.float32))
    mfs = jax.nn.sigmoid(attn_mix_w).reshape(L_LAYERS, 1, 1)

    src0, dst0, vals0 = _prep_edges(edge_index[0], adj_vals[0])
    src1, dst1, vals1 = _prep_edges(edge_index[1], adj_vals[1])
    anchor0, h0att_split, w_i = _fc_att(
        x[0], W_fc[0], b_fc[0].reshape(1, H), Ts[0], attn_fc_w[0],
        attn_fc_b[0].reshape(1, H), mfs[0])
    h1, h1_split = _fc(x[1], W_fc[1], b_fc[1].reshape(1, H))
    anchor1 = h1
    h0 = None
    for i in range(L_LAYERS):
        hi0 = _spmm_sc(h0att_split, src0, dst0, vals0)
        hi1 = _spmm_sc(h1_split, src1, dst1, vals1)
        w_cur = w_i
        if i + 1 < L_LAYERS:
            h0att_split, w_i = _tail_att(
                hi0, anchor0, Ms[i], Ts[i + 1], attn_fc_w[i + 1],
                attn_fc_b[i + 1].reshape(1, H), mfs[i + 1])
        else:
            h0 = _tail(hi0, anchor0, Ms[i])
        h1, h1_split = _tail_mul(hi1, anchor1, Ms[i], w_cur)

    return _head(h0, h1, W_out, b_out.reshape(1, C_OUT))
